# Initial kernel scaffold; baseline (speedup 1.0000x reference)
import jax
import jax.numpy as jnp
from jax.experimental import pallas as pl


def kernel(b_s, b_ns, b_r, b_d, W1, b1, W2, b2, TW1, Tb1, TW2, Tb2, b_a):
    raise NotImplementedError("write your pallas kernel here")


# speedup vs baseline: 32.0376x; 32.0376x over previous
import functools

import jax
import jax.numpy as jnp
from jax.experimental import pallas as pl

V_MIN = -10.0
V_MAX = 10.0
N_ATOM = 51
GAMMA = 0.99
V_STEP = (V_MAX - V_MIN) / (N_ATOM - 1)
GP = 56
NEG = -1e30


def _c51_block(xT_ref, nT_ref, r_ref, d_ref, a_ref,
               W1T_ref, b1_ref, W2T_ref, b2_ref,
               TW1T_ref, Tb1_ref, TW2T_ref, Tb2_ref,
               out_ref, *, n_act):
    f32 = jnp.float32

    hn = jnp.maximum(
        jnp.dot(TW1T_ref[...], nT_ref[...], preferred_element_type=f32)
        + Tb1_ref[...], 0.0)
    lt = jnp.dot(TW2T_ref[...], hn, preferred_element_type=f32) + Tb2_ref[...]

    h = jnp.maximum(
        jnp.dot(W1T_ref[...], xT_ref[...], preferred_element_type=f32)
        + b1_ref[...], 0.0)
    le = jnp.dot(W2T_ref[...], h, preferred_element_type=f32) + b2_ref[...]

    jio = jax.lax.broadcasted_iota(jnp.int32, (GP, 1), 0).astype(f32)
    vcol = jio * f32(V_STEP) + f32(V_MIN)

    means = []
    for a in range(n_act):
        g = lt[a * GP:(a + 1) * GP, :]
        m = jnp.max(g, axis=0, keepdims=True)
        e = jnp.exp(g - m)
        s = jnp.sum(e, axis=0, keepdims=True)
        w = jnp.sum(e * vcol, axis=0, keepdims=True)
        means.append(w / s)

    best = means[0]
    bidx = jnp.zeros_like(best)
    for a in range(1, n_act):
        gt = means[a] > best
        best = jnp.where(gt, means[a], best)
        bidx = jnp.where(gt, f32(a), bidx)

    cl = jnp.where(bidx == 0.0, lt[0:GP, :], 0.0)
    for a in range(1, n_act):
        cl = cl + jnp.where(bidx == f32(a), lt[a * GP:(a + 1) * GP, :], 0.0)
    m = jnp.max(cl, axis=0, keepdims=True)
    e = jnp.exp(cl - m)
    qn = e / jnp.sum(e, axis=0, keepdims=True)

    av = a_ref[...]
    ce = jnp.where(av == 0, le[0:GP, :], 0.0)
    for a in range(1, n_act):
        ce = ce + jnp.where(av == a, le[a * GP:(a + 1) * GP, :], 0.0)
    m = jnp.max(ce, axis=0, keepdims=True)
    e = jnp.exp(ce - m)
    qe = e / jnp.sum(e, axis=0, keepdims=True)
    nl = -jnp.log(qe + f32(1e-8))

    rrow = r_ref[...]
    scale = f32(GAMMA) * (1.0 - d_ref[...])
    qt = jnp.zeros_like(qn)
    for z in range(N_ATOM):
        vz = V_MIN + V_STEP * z
        tz = jnp.clip(rrow + scale * f32(vz), f32(V_MIN), f32(V_MAX))
        posz = (tz - f32(V_MIN)) / f32(V_STEP)
        lbz = jnp.floor(posz)
        ubz = jnp.ceil(posz)
        qz = qn[z:z + 1, :]
        w_l = qz * (ubz - posz)
        w_u = qz * (posz - lbz)
        qt = qt + jnp.where(jio == lbz, w_l, jnp.where(jio == ubz, w_u, 0.0))

    bl = jnp.sum(qt * nl, keepdims=True)

    @pl.when(pl.program_id(0) == 0)
    def _init():
        out_ref[...] = jnp.zeros_like(out_ref)

    out_ref[...] += bl


def kernel(b_s, b_ns, b_r, b_d, W1, b1, W2, b2, TW1, Tb1, TW2, Tb2, b_a):
    B, S = b_s.shape
    H = W1.shape[1]
    n_act = W2.shape[1] // N_ATOM

    def pad_head(W, b):
        Wp = jnp.pad(W.reshape(H, n_act, N_ATOM), ((0, 0), (0, 0), (0, GP - N_ATOM)))
        bp = jnp.pad(b.reshape(n_act, N_ATOM), ((0, 0), (0, GP - N_ATOM)),
                     constant_values=NEG)
        return Wp.reshape(H, n_act * GP).T, bp.reshape(n_act * GP, 1)

    W2T, b2c = pad_head(W2, b2)
    TW2T, Tb2c = pad_head(TW2, Tb2)
    xT = b_s.T
    nT = b_ns.T
    r2 = b_r.reshape(1, B)
    d2 = b_d.reshape(1, B)
    a2 = b_a.reshape(1, B).astype(jnp.int32)

    BB = 512
    grid = B // BB
    OD = n_act * GP

    out = pl.pallas_call(
        functools.partial(_c51_block, n_act=n_act),
        grid=(grid,),
        in_specs=[
            pl.BlockSpec((S, BB), lambda i: (0, i)),
            pl.BlockSpec((S, BB), lambda i: (0, i)),
            pl.BlockSpec((1, BB), lambda i: (0, i)),
            pl.BlockSpec((1, BB), lambda i: (0, i)),
            pl.BlockSpec((1, BB), lambda i: (0, i)),
            pl.BlockSpec((H, S), lambda i: (0, 0)),
            pl.BlockSpec((H, 1), lambda i: (0, 0)),
            pl.BlockSpec((OD, H), lambda i: (0, 0)),
            pl.BlockSpec((OD, 1), lambda i: (0, 0)),
            pl.BlockSpec((H, S), lambda i: (0, 0)),
            pl.BlockSpec((H, 1), lambda i: (0, 0)),
            pl.BlockSpec((OD, H), lambda i: (0, 0)),
            pl.BlockSpec((OD, 1), lambda i: (0, 0)),
        ],
        out_specs=pl.BlockSpec((1, 1), lambda i: (0, 0)),
        out_shape=jax.ShapeDtypeStruct((1, 1), jnp.float32),
    )(xT, nT, r2, d2, a2,
      W1.T, b1.reshape(H, 1), W2T, b2c,
      TW1.T, Tb1.reshape(H, 1), TW2T, Tb2c)
    return out[0, 0] / (B * N_ATOM)


# speedup vs baseline: 33.6861x; 1.0515x over previous
import functools

import jax
import jax.numpy as jnp
from jax.experimental import pallas as pl

V_MIN = -10.0
V_MAX = 10.0
N_ATOM = 51
GAMMA = 0.99
V_STEP = (V_MAX - V_MIN) / (N_ATOM - 1)
GP = 56
NEG = -1e30


def _tree_select(groups, bidx):
    level = list(groups)
    bit = 0
    while len(level) > 1:
        mask = ((bidx >> bit) & 1) == 1
        level = [jnp.where(mask, level[2 * k + 1], level[2 * k])
                 for k in range(len(level) // 2)]
        bit += 1
    return level[0]


def _c51_block(xT_ref, nT_ref, r_ref, d_ref, a_ref,
               W1T_ref, b1_ref, W2T_ref, b2_ref,
               TW1T_ref, Tb1_ref, TW2T_ref, Tb2_ref,
               P_ref, M_ref,
               out_ref, *, n_act):
    f32 = jnp.float32

    hn = jnp.maximum(
        jnp.dot(TW1T_ref[...], nT_ref[...], preferred_element_type=f32)
        + Tb1_ref[...], 0.0)
    lt = jnp.dot(TW2T_ref[...], hn, preferred_element_type=f32) + Tb2_ref[...]

    h = jnp.maximum(
        jnp.dot(W1T_ref[...], xT_ref[...], preferred_element_type=f32)
        + b1_ref[...], 0.0)
    le = jnp.dot(W2T_ref[...], h, preferred_element_type=f32) + b2_ref[...]

    maxes = [jnp.max(lt[a * GP:(a + 1) * GP, :], axis=0, keepdims=True)
             for a in range(n_act)]
    mstack = jnp.concatenate(maxes, axis=0)
    mfull = jnp.dot(P_ref[...], mstack, preferred_element_type=f32)
    e = jnp.exp(lt - mfull)
    S2 = jnp.dot(M_ref[...], e, preferred_element_type=f32)
    s_rows = S2[0:n_act, :]
    w_rows = S2[n_act:2 * n_act, :]
    means = w_rows / s_rows

    mmax = jnp.max(means, axis=0, keepdims=True)
    i16 = jax.lax.broadcasted_iota(jnp.int32, (n_act, 1), 0)
    bidx = jnp.min(jnp.where(means == mmax, i16, n_act), axis=0,
                   keepdims=True)

    e_sel = _tree_select([e[a * GP:(a + 1) * GP, :] for a in range(n_act)],
                         bidx)
    s_sel = _tree_select([s_rows[a:a + 1, :] for a in range(n_act)], bidx)
    qn = e_sel / s_sel

    aidx = a_ref[...]
    ce = _tree_select([le[a * GP:(a + 1) * GP, :] for a in range(n_act)],
                      aidx)
    m = jnp.max(ce, axis=0, keepdims=True)
    ex = jnp.exp(ce - m)
    qe = ex / jnp.sum(ex, axis=0, keepdims=True)
    nl = -jnp.log(qe + f32(1e-8))

    jio = jax.lax.broadcasted_iota(jnp.int32, (GP, 1), 0).astype(f32)
    vcol = jio * f32(V_STEP) + f32(V_MIN)
    scale = f32(GAMMA) * (1.0 - d_ref[...])
    pos = (jnp.clip(r_ref[...] + scale * vcol, f32(V_MIN), f32(V_MAX))
           - f32(V_MIN)) / f32(V_STEP)
    lb = jnp.floor(pos)
    ub = jnp.ceil(pos)
    wl = qn * (ub - pos)
    wu = qn * (pos - lb)
    qt = jnp.zeros_like(qn)
    for z in range(N_ATOM):
        lbz = lb[z:z + 1, :]
        ubz = ub[z:z + 1, :]
        qt = qt + jnp.where(jio == lbz, wl[z:z + 1, :],
                            jnp.where(jio == ubz, wu[z:z + 1, :], 0.0))

    bl = jnp.sum(qt * nl, keepdims=True)

    @pl.when(pl.program_id(0) == 0)
    def _init():
        out_ref[...] = jnp.zeros_like(out_ref)

    out_ref[...] += bl


def kernel(b_s, b_ns, b_r, b_d, W1, b1, W2, b2, TW1, Tb1, TW2, Tb2, b_a):
    B, S = b_s.shape
    H = W1.shape[1]
    n_act = W2.shape[1] // N_ATOM
    OD = n_act * GP

    def pad_head(W, b):
        Wp = jnp.pad(W.reshape(H, n_act, N_ATOM), ((0, 0), (0, 0), (0, GP - N_ATOM)))
        bp = jnp.pad(b.reshape(n_act, N_ATOM), ((0, 0), (0, GP - N_ATOM)),
                     constant_values=NEG)
        return Wp.reshape(H, OD).T, bp.reshape(OD, 1)

    W2T, b2c = pad_head(W2, b2)
    TW2T, Tb2c = pad_head(TW2, Tb2)
    xT = b_s.T
    nT = b_ns.T
    r2 = b_r.reshape(1, B)
    d2 = b_d.reshape(1, B)
    a2 = b_a.reshape(1, B).astype(jnp.int32)

    grp = jnp.arange(OD) // GP
    atom = jnp.arange(OD) % GP
    vvals = (atom * V_STEP + V_MIN).astype(jnp.float32)
    Pmat = (grp[:, None] == jnp.arange(n_act)[None, :]).astype(jnp.float32)
    ind = (jnp.arange(n_act)[:, None] == grp[None, :]).astype(jnp.float32)
    Mmat = jnp.concatenate([ind, ind * vvals[None, :]], axis=0)

    BB = 512
    grid = B // BB

    out = pl.pallas_call(
        functools.partial(_c51_block, n_act=n_act),
        grid=(grid,),
        in_specs=[
            pl.BlockSpec((S, BB), lambda i: (0, i)),
            pl.BlockSpec((S, BB), lambda i: (0, i)),
            pl.BlockSpec((1, BB), lambda i: (0, i)),
            pl.BlockSpec((1, BB), lambda i: (0, i)),
            pl.BlockSpec((1, BB), lambda i: (0, i)),
            pl.BlockSpec((H, S), lambda i: (0, 0)),
            pl.BlockSpec((H, 1), lambda i: (0, 0)),
            pl.BlockSpec((OD, H), lambda i: (0, 0)),
            pl.BlockSpec((OD, 1), lambda i: (0, 0)),
            pl.BlockSpec((H, S), lambda i: (0, 0)),
            pl.BlockSpec((H, 1), lambda i: (0, 0)),
            pl.BlockSpec((OD, H), lambda i: (0, 0)),
            pl.BlockSpec((OD, 1), lambda i: (0, 0)),
            pl.BlockSpec((OD, n_act), lambda i: (0, 0)),
            pl.BlockSpec((2 * n_act, OD), lambda i: (0, 0)),
        ],
        out_specs=pl.BlockSpec((1, 1), lambda i: (0, 0)),
        out_shape=jax.ShapeDtypeStruct((1, 1), jnp.float32),
    )(xT, nT, r2, d2, a2,
      W1.T, b1.reshape(H, 1), W2T, b2c,
      TW1.T, Tb1.reshape(H, 1), TW2T, Tb2c,
      Pmat, Mmat)
    return out[0, 0] / (B * N_ATOM)


# speedup vs baseline: 41.9750x; 1.2461x over previous
import functools

import jax
import jax.numpy as jnp
from jax import lax
from jax.experimental import pallas as pl
from jax.experimental.pallas import tpu as pltpu
from jax.experimental.pallas import tpu_sc as plsc

V_MIN = -10.0
V_MAX = 10.0
N_ATOM = 51
GAMMA = 0.99
V_STEP = (V_MAX - V_MIN) / (N_ATOM - 1)
GP = 56
NEG = -1e30
NW = 32
SC_CHUNK = 512


def _tree_select(groups, bidx):
    level = list(groups)
    bit = 0
    while len(level) > 1:
        mask = ((bidx >> bit) & 1) == 1
        level = [jnp.where(mask, level[2 * k + 1], level[2 * k])
                 for k in range(len(level) // 2)]
        bit += 1
    return level[0]


def _c51_tc_block(xT_ref, nT_ref, a_ref,
                  W1T_ref, b1_ref, W2T_ref, b2_ref,
                  TW1T_ref, Tb1_ref, TW2T_ref, Tb2_ref,
                  P_ref, M_ref,
                  qn_ref, nl_ref, *, n_act):
    f32 = jnp.float32

    hn = jnp.maximum(
        jnp.dot(TW1T_ref[...], nT_ref[...], preferred_element_type=f32)
        + Tb1_ref[...], 0.0)
    lt = jnp.dot(TW2T_ref[...], hn, preferred_element_type=f32) + Tb2_ref[...]

    h = jnp.maximum(
        jnp.dot(W1T_ref[...], xT_ref[...], preferred_element_type=f32)
        + b1_ref[...], 0.0)
    le = jnp.dot(W2T_ref[...], h, preferred_element_type=f32) + b2_ref[...]

    maxes = [jnp.max(lt[a * GP:(a + 1) * GP, :], axis=0, keepdims=True)
             for a in range(n_act)]
    mstack = jnp.concatenate(maxes, axis=0)
    mfull = jnp.dot(P_ref[...], mstack, preferred_element_type=f32)
    e = jnp.exp(lt - mfull)
    S2 = jnp.dot(M_ref[...], e, preferred_element_type=f32)
    s_rows = S2[0:n_act, :]
    w_rows = S2[n_act:2 * n_act, :]
    means = w_rows / s_rows

    mmax = jnp.max(means, axis=0, keepdims=True)
    i16 = jax.lax.broadcasted_iota(jnp.int32, (n_act, 1), 0)
    bidx = jnp.min(jnp.where(means == mmax, i16, n_act), axis=0,
                   keepdims=True)

    e_sel = _tree_select([e[a * GP:(a + 1) * GP, :] for a in range(n_act)],
                         bidx)
    s_sel = _tree_select([s_rows[a:a + 1, :] for a in range(n_act)], bidx)
    qn_ref[...] = e_sel / s_sel

    aidx = a_ref[...]
    ce = _tree_select([le[a * GP:(a + 1) * GP, :] for a in range(n_act)],
                      aidx)
    m = jnp.max(ce, axis=0, keepdims=True)
    ex = jnp.exp(ce - m)
    qe = ex / jnp.sum(ex, axis=0, keepdims=True)
    nl_ref[...] = -jnp.log(qe + f32(1e-8))


def _sc_project(qn_hbm, nl_hbm, r_hbm, d_hbm, out_hbm,
                qn_v, nl_v, r_v, d_v, acc_v):
    f32 = jnp.float32
    wid = lax.axis_index("s") * 2 + lax.axis_index("c")
    rows_per_w = qn_hbm.shape[1] // NW
    base = wid * rows_per_w
    lane = lax.broadcasted_iota(jnp.int32, (16,), 0)

    acc_v[...] = jnp.zeros((16,), f32)
    for c in range(rows_per_w // SC_CHUNK):
        c0 = base + c * SC_CHUNK
        pltpu.sync_copy(qn_hbm.at[0:GP, pl.ds(c0, SC_CHUNK)], qn_v)
        pltpu.sync_copy(nl_hbm.at[0:GP, pl.ds(c0, SC_CHUNK)], nl_v)
        pltpu.sync_copy(r_hbm.at[pl.ds(c0, SC_CHUNK)], r_v)
        pltpu.sync_copy(d_hbm.at[pl.ds(c0, SC_CHUNK)], d_v)

        def grp(g, acc):
            r0 = g * 16
            rr = r_v[pl.ds(r0, 16)]
            sc = f32(GAMMA) * (1.0 - d_v[pl.ds(r0, 16)])
            ridx = r0 + lane
            for z in range(N_ATOM):
                vz = f32(V_MIN + V_STEP * z)
                tz = jnp.clip(rr + sc * vz, f32(V_MIN), f32(V_MAX))
                pos = (tz - f32(V_MIN)) / f32(V_STEP)
                lb_i = pos.astype(jnp.int32)
                frac = pos - lb_i.astype(f32)
                hi = (frac > 0.0)
                qz = qn_v[z, pl.ds(r0, 16)]
                wl = qz * jnp.where(hi, 1.0 - frac, 0.0)
                wu = qz * frac
                ub_i = lb_i + hi.astype(jnp.int32)
                g_l = plsc.load_gather(nl_v, [lb_i, ridx])
                g_u = plsc.load_gather(nl_v, [ub_i, ridx])
                acc = acc + wl * g_l + wu * g_u
            return acc

        acc_v[...] += lax.fori_loop(0, SC_CHUNK // 16, grp,
                                    jnp.zeros((16,), f32))
    pltpu.sync_copy(acc_v, out_hbm.at[pl.ds(wid * 16, 16)])


def kernel(b_s, b_ns, b_r, b_d, W1, b1, W2, b2, TW1, Tb1, TW2, Tb2, b_a):
    B, S = b_s.shape
    H = W1.shape[1]
    n_act = W2.shape[1] // N_ATOM
    OD = n_act * GP

    def pad_head(W, b):
        Wp = jnp.pad(W.reshape(H, n_act, N_ATOM), ((0, 0), (0, 0), (0, GP - N_ATOM)))
        bp = jnp.pad(b.reshape(n_act, N_ATOM), ((0, 0), (0, GP - N_ATOM)),
                     constant_values=NEG)
        return Wp.reshape(H, OD).T, bp.reshape(OD, 1)

    W2T, b2c = pad_head(W2, b2)
    TW2T, Tb2c = pad_head(TW2, Tb2)
    xT = b_s.T
    nT = b_ns.T
    a2 = b_a.reshape(1, B).astype(jnp.int32)

    grp = jnp.arange(OD) // GP
    atom = jnp.arange(OD) % GP
    vvals = (atom * V_STEP + V_MIN).astype(jnp.float32)
    Pmat = (grp[:, None] == jnp.arange(n_act)[None, :]).astype(jnp.float32)
    ind = (jnp.arange(n_act)[:, None] == grp[None, :]).astype(jnp.float32)
    Mmat = jnp.concatenate([ind, ind * vvals[None, :]], axis=0)

    BB = 512
    grid = B // BB

    qnT, nlT = pl.pallas_call(
        functools.partial(_c51_tc_block, n_act=n_act),
        grid=(grid,),
        in_specs=[
            pl.BlockSpec((S, BB), lambda i: (0, i)),
            pl.BlockSpec((S, BB), lambda i: (0, i)),
            pl.BlockSpec((1, BB), lambda i: (0, i)),
            pl.BlockSpec((H, S), lambda i: (0, 0)),
            pl.BlockSpec((H, 1), lambda i: (0, 0)),
            pl.BlockSpec((OD, H), lambda i: (0, 0)),
            pl.BlockSpec((OD, 1), lambda i: (0, 0)),
            pl.BlockSpec((H, S), lambda i: (0, 0)),
            pl.BlockSpec((H, 1), lambda i: (0, 0)),
            pl.BlockSpec((OD, H), lambda i: (0, 0)),
            pl.BlockSpec((OD, 1), lambda i: (0, 0)),
            pl.BlockSpec((OD, n_act), lambda i: (0, 0)),
            pl.BlockSpec((2 * n_act, OD), lambda i: (0, 0)),
        ],
        out_specs=[
            pl.BlockSpec((GP, BB), lambda i: (0, i)),
            pl.BlockSpec((GP, BB), lambda i: (0, i)),
        ],
        out_shape=[
            jax.ShapeDtypeStruct((GP, B), jnp.float32),
            jax.ShapeDtypeStruct((GP, B), jnp.float32),
        ],
    )(xT, nT, a2,
      W1.T, b1.reshape(H, 1), W2T, b2c,
      TW1.T, Tb1.reshape(H, 1), TW2T, Tb2c,
      Pmat, Mmat)

    sc_loss = functools.partial(
        pl.kernel,
        mesh=plsc.VectorSubcoreMesh(core_axis_name="c", subcore_axis_name="s"),
        compiler_params=pltpu.CompilerParams(needs_layout_passes=False),
        out_type=jax.ShapeDtypeStruct((NW * 16,), jnp.float32),
        scratch_types=[
            pltpu.VMEM((GP, SC_CHUNK), jnp.float32),
            pltpu.VMEM((GP, SC_CHUNK), jnp.float32),
            pltpu.VMEM((SC_CHUNK,), jnp.float32),
            pltpu.VMEM((SC_CHUNK,), jnp.float32),
            pltpu.VMEM((16,), jnp.float32),
        ],
    )(_sc_project)

    partials = sc_loss(qnT, nlT, b_r, b_d)
    return jnp.sum(partials) / (B * N_ATOM)


# speedup vs baseline: 46.8360x; 1.1158x over previous
import functools

import jax
import jax.numpy as jnp
from jax import lax
from jax.experimental import pallas as pl
from jax.experimental.pallas import tpu as pltpu
from jax.experimental.pallas import tpu_sc as plsc

V_MIN = -10.0
V_MAX = 10.0
N_ATOM = 51
GAMMA = 0.99
V_STEP = (V_MAX - V_MIN) / (N_ATOM - 1)
GP = 56
NEG = -1e30
NW = 32
SC_CHUNK = 512
NSPLIT = 2


def _tree_select(groups, bidx):
    level = list(groups)
    bit = 0
    while len(level) > 1:
        mask = ((bidx >> bit) & 1) == 1
        level = [jnp.where(mask, level[2 * k + 1], level[2 * k])
                 for k in range(len(level) // 2)]
        bit += 1
    return level[0]


def _c51_tc_block(x_ref, n_ref, a_ref,
                  W1_ref, b1_ref, W2T_ref, b2_ref,
                  TW1_ref, Tb1_ref, TW2T_ref, Tb2_ref,
                  P_ref, M_ref,
                  qn_ref, nl_ref, *, n_act):
    f32 = jnp.float32
    dn_nat = (((1,), (0,)), ((), ()))
    dn_tr = (((1,), (1,)), ((), ()))

    hn = jnp.maximum(
        lax.dot_general(n_ref[...], TW1_ref[...], dn_nat,
                        preferred_element_type=f32) + Tb1_ref[...], 0.0)
    lt = lax.dot_general(TW2T_ref[...], hn, dn_tr,
                         preferred_element_type=f32) + Tb2_ref[...]

    h = jnp.maximum(
        lax.dot_general(x_ref[...], W1_ref[...], dn_nat,
                        preferred_element_type=f32) + b1_ref[...], 0.0)
    le = lax.dot_general(W2T_ref[...], h, dn_tr,
                         preferred_element_type=f32) + b2_ref[...]

    maxes = [jnp.max(lt[a * GP:(a + 1) * GP, :], axis=0, keepdims=True)
             for a in range(n_act)]
    mstack = jnp.concatenate(maxes, axis=0)
    mfull = jnp.dot(P_ref[...], mstack, preferred_element_type=f32)
    e = jnp.exp(lt - mfull)
    S2 = jnp.dot(M_ref[...], e, preferred_element_type=f32)
    s_rows = S2[0:n_act, :]
    w_rows = S2[n_act:2 * n_act, :]
    means = w_rows / s_rows

    mmax = jnp.max(means, axis=0, keepdims=True)
    i16 = jax.lax.broadcasted_iota(jnp.int32, (n_act, 1), 0)
    bidx = jnp.min(jnp.where(means == mmax, i16, n_act), axis=0,
                   keepdims=True)

    e_sel = _tree_select([e[a * GP:(a + 1) * GP, :] for a in range(n_act)],
                         bidx)
    s_sel = _tree_select([s_rows[a:a + 1, :] for a in range(n_act)], bidx)
    qn_ref[...] = e_sel / s_sel

    aidx = a_ref[...]
    ce = _tree_select([le[a * GP:(a + 1) * GP, :] for a in range(n_act)],
                      aidx)
    m = jnp.max(ce, axis=0, keepdims=True)
    ex = jnp.exp(ce - m)
    qe = ex / jnp.sum(ex, axis=0, keepdims=True)
    nl_ref[...] = -jnp.log(qe + f32(1e-8))


def _sc_project(qn_hbm, nl_hbm, r_hbm, d_hbm, out_hbm,
                qn_v, nl_v, r_v, d_v, acc_v):
    f32 = jnp.float32
    wid = lax.axis_index("s") * 2 + lax.axis_index("c")
    rows_per_w = qn_hbm.shape[1] // NW
    base = wid * rows_per_w
    lane = lax.broadcasted_iota(jnp.int32, (16,), 0)

    acc_v[...] = jnp.zeros((16,), f32)
    for c in range(rows_per_w // SC_CHUNK):
        c0 = base + c * SC_CHUNK
        pltpu.sync_copy(qn_hbm.at[0:GP, pl.ds(c0, SC_CHUNK)], qn_v)
        pltpu.sync_copy(nl_hbm.at[0:GP, pl.ds(c0, SC_CHUNK)], nl_v)
        pltpu.sync_copy(r_hbm.at[pl.ds(c0, SC_CHUNK)], r_v)
        pltpu.sync_copy(d_hbm.at[pl.ds(c0, SC_CHUNK)], d_v)

        def grp(g, accs):
            acc_a, acc_b = accs
            r0 = g * 16
            rr = r_v[pl.ds(r0, 16)]
            sc = f32(GAMMA) * (1.0 - d_v[pl.ds(r0, 16)])
            p0 = (rr - f32(V_MIN) + sc * f32(V_MIN)) * f32(1.0 / V_STEP)
            ridx = r0 + lane
            for z in range(N_ATOM):
                pos = jnp.clip(p0 + sc * f32(z), f32(0.0), f32(N_ATOM - 1))
                lb_i = pos.astype(jnp.int32)
                frac = pos - lb_i.astype(f32)
                hi = (frac > 0.0)
                qz = qn_v[z, pl.ds(r0, 16)]
                wl = qz * jnp.where(hi, 1.0 - frac, 0.0)
                wu = qz * frac
                ub_i = lb_i + hi.astype(jnp.int32)
                g_l = plsc.load_gather(nl_v, [lb_i, ridx])
                g_u = plsc.load_gather(nl_v, [ub_i, ridx])
                acc_a = acc_a + wl * g_l
                acc_b = acc_b + wu * g_u
            return acc_a, acc_b

        pa, pb = lax.fori_loop(0, SC_CHUNK // 16, grp,
                               (jnp.zeros((16,), f32), jnp.zeros((16,), f32)))
        acc_v[...] += pa + pb
    pltpu.sync_copy(acc_v, out_hbm.at[pl.ds(wid * 16, 16)])


def kernel(b_s, b_ns, b_r, b_d, W1, b1, W2, b2, TW1, Tb1, TW2, Tb2, b_a):
    B, S = b_s.shape
    H = W1.shape[1]
    n_act = W2.shape[1] // N_ATOM
    OD = n_act * GP

    def pad_head(W, b):
        Wp = jnp.pad(W.reshape(H, n_act, N_ATOM), ((0, 0), (0, 0), (0, GP - N_ATOM)))
        bp = jnp.pad(b.reshape(n_act, N_ATOM), ((0, 0), (0, GP - N_ATOM)),
                     constant_values=NEG)
        return Wp.reshape(H, OD).T, bp.reshape(OD, 1)

    W2T, b2c = pad_head(W2, b2)
    TW2T, Tb2c = pad_head(TW2, Tb2)
    a2 = b_a.reshape(1, B).astype(jnp.int32)

    grp = jnp.arange(OD) // GP
    atom = jnp.arange(OD) % GP
    vvals = (atom * V_STEP + V_MIN).astype(jnp.float32)
    Pmat = (grp[:, None] == jnp.arange(n_act)[None, :]).astype(jnp.float32)
    ind = (jnp.arange(n_act)[:, None] == grp[None, :]).astype(jnp.float32)
    Mmat = jnp.concatenate([ind, ind * vvals[None, :]], axis=0)

    BB = 512
    BH = B // NSPLIT
    grid = BH // BB

    tc_call = pl.pallas_call(
        functools.partial(_c51_tc_block, n_act=n_act),
        grid=(grid,),
        in_specs=[
            pl.BlockSpec((BB, S), lambda i: (i, 0)),
            pl.BlockSpec((BB, S), lambda i: (i, 0)),
            pl.BlockSpec((1, BB), lambda i: (0, i)),
            pl.BlockSpec((S, H), lambda i: (0, 0)),
            pl.BlockSpec((1, H), lambda i: (0, 0)),
            pl.BlockSpec((OD, H), lambda i: (0, 0)),
            pl.BlockSpec((OD, 1), lambda i: (0, 0)),
            pl.BlockSpec((S, H), lambda i: (0, 0)),
            pl.BlockSpec((1, H), lambda i: (0, 0)),
            pl.BlockSpec((OD, H), lambda i: (0, 0)),
            pl.BlockSpec((OD, 1), lambda i: (0, 0)),
            pl.BlockSpec((OD, n_act), lambda i: (0, 0)),
            pl.BlockSpec((2 * n_act, OD), lambda i: (0, 0)),
        ],
        out_specs=[
            pl.BlockSpec((GP, BB), lambda i: (0, i)),
            pl.BlockSpec((GP, BB), lambda i: (0, i)),
        ],
        out_shape=[
            jax.ShapeDtypeStruct((GP, BH), jnp.float32),
            jax.ShapeDtypeStruct((GP, BH), jnp.float32),
        ],
    )

    sc_loss = functools.partial(
        pl.kernel,
        mesh=plsc.VectorSubcoreMesh(core_axis_name="c", subcore_axis_name="s"),
        compiler_params=pltpu.CompilerParams(needs_layout_passes=False),
        out_type=jax.ShapeDtypeStruct((NW * 16,), jnp.float32),
        scratch_types=[
            pltpu.VMEM((GP, SC_CHUNK), jnp.float32),
            pltpu.VMEM((GP, SC_CHUNK), jnp.float32),
            pltpu.VMEM((SC_CHUNK,), jnp.float32),
            pltpu.VMEM((SC_CHUNK,), jnp.float32),
            pltpu.VMEM((16,), jnp.float32),
        ],
    )(_sc_project)

    total = jnp.zeros((), jnp.float32)
    for p in range(NSPLIT):
        lo = p * BH
        qnT, nlT = tc_call(
            lax.slice_in_dim(b_s, lo, lo + BH, axis=0),
            lax.slice_in_dim(b_ns, lo, lo + BH, axis=0),
            lax.slice_in_dim(a2, lo, lo + BH, axis=1),
            W1, b1.reshape(1, H), W2T, b2c,
            TW1, Tb1.reshape(1, H), TW2T, Tb2c,
            Pmat, Mmat)
        partials = sc_loss(qnT, nlT,
                           lax.slice_in_dim(b_r, lo, lo + BH, axis=0),
                           lax.slice_in_dim(b_d, lo, lo + BH, axis=0))
        total = total + jnp.sum(partials)
    return total / (B * N_ATOM)


# speedup vs baseline: 55.9975x; 1.1956x over previous
import functools

import jax
import jax.numpy as jnp
from jax import lax
from jax.experimental import pallas as pl
from jax.experimental.pallas import tpu as pltpu
from jax.experimental.pallas import tpu_sc as plsc

V_MIN = -10.0
V_MAX = 10.0
N_ATOM = 51
GAMMA = 0.99
V_STEP = (V_MAX - V_MIN) / (N_ATOM - 1)
GP = 56
NEG = -1e30
NW = 32
SC_CHUNK = 512
NSPLIT = 2


def _tree_select(groups, bidx):
    level = list(groups)
    bit = 0
    while len(level) > 1:
        mask = ((bidx >> bit) & 1) == 1
        level = [jnp.where(mask, level[2 * k + 1], level[2 * k])
                 for k in range(len(level) // 2)]
        bit += 1
    return level[0]


def _c51_tc_block(x_ref, n_ref, a_ref,
                  W1_ref, b1_ref, W2T_ref, b2_ref,
                  TW1_ref, Tb1_ref, TW2T_ref, Tb2_ref,
                  P_ref, M_ref,
                  qn_ref, nl_ref, *, n_act):
    f32 = jnp.float32
    dn_nat = (((1,), (0,)), ((), ()))
    dn_tr = (((1,), (1,)), ((), ()))

    hn = jnp.maximum(
        lax.dot_general(n_ref[...], TW1_ref[...], dn_nat,
                        preferred_element_type=f32) + Tb1_ref[...], 0.0)
    lt = lax.dot_general(TW2T_ref[...], hn, dn_tr,
                         preferred_element_type=f32) + Tb2_ref[...]

    h = jnp.maximum(
        lax.dot_general(x_ref[...], W1_ref[...], dn_nat,
                        preferred_element_type=f32) + b1_ref[...], 0.0)
    le = lax.dot_general(W2T_ref[...], h, dn_tr,
                         preferred_element_type=f32) + b2_ref[...]

    maxes = [jnp.max(lt[a * GP:(a + 1) * GP, :], axis=0, keepdims=True)
             for a in range(n_act)]
    mstack = jnp.concatenate(maxes, axis=0)
    mfull = jnp.dot(P_ref[...], mstack, preferred_element_type=f32)
    e = jnp.exp(lt - mfull)
    S2 = jnp.dot(M_ref[...], e, preferred_element_type=f32)
    s_rows = S2[0:n_act, :]
    w_rows = S2[n_act:2 * n_act, :]
    means = w_rows / s_rows

    mmax = jnp.max(means, axis=0, keepdims=True)
    i16 = jax.lax.broadcasted_iota(jnp.int32, (n_act, 1), 0)
    bidx = jnp.min(jnp.where(means == mmax, i16, n_act), axis=0,
                   keepdims=True)

    e_sel = _tree_select([e[a * GP:(a + 1) * GP, :] for a in range(n_act)],
                         bidx)
    s_sel = _tree_select([s_rows[a:a + 1, :] for a in range(n_act)], bidx)
    qn_ref[...] = e_sel / s_sel

    aidx = a_ref[...]
    ce = _tree_select([le[a * GP:(a + 1) * GP, :] for a in range(n_act)],
                      aidx)
    m = jnp.max(ce, axis=0, keepdims=True)
    ex = jnp.exp(ce - m)
    qe = ex / jnp.sum(ex, axis=0, keepdims=True)
    nl_ref[...] = -jnp.log(qe + f32(1e-8))


def _sc_project(qn_hbm, nl_hbm, r_hbm, d_hbm, out_hbm,
                qn_v, nl_v, r_v, d_v, acc_v):
    f32 = jnp.float32
    wid = lax.axis_index("s") * 2 + lax.axis_index("c")
    rows_per_w = qn_hbm.shape[1] // NW
    base = wid * rows_per_w
    lane = lax.broadcasted_iota(jnp.int32, (16,), 0)

    acc_v[...] = jnp.zeros((16,), f32)
    for c in range(rows_per_w // SC_CHUNK):
        c0 = base + c * SC_CHUNK
        pltpu.sync_copy(qn_hbm.at[0:GP, pl.ds(c0, SC_CHUNK)], qn_v)
        pltpu.sync_copy(nl_hbm.at[0:GP, pl.ds(c0, SC_CHUNK)], nl_v)
        pltpu.sync_copy(r_hbm.at[pl.ds(c0, SC_CHUNK)], r_v)
        pltpu.sync_copy(d_hbm.at[pl.ds(c0, SC_CHUNK)], d_v)

        def grp(g, accs):
            acc_a, acc_b = accs
            r0 = g * 16
            rr = r_v[pl.ds(r0, 16)]
            sc = f32(GAMMA) * (1.0 - d_v[pl.ds(r0, 16)])
            p0 = (rr - f32(V_MIN) + sc * f32(V_MIN)) * f32(1.0 / V_STEP)
            ridx = r0 + lane
            for z in range(N_ATOM):
                pos = jnp.clip(p0 + sc * f32(z), f32(0.0), f32(N_ATOM - 1))
                lb_i = pos.astype(jnp.int32)
                frac = pos - lb_i.astype(f32)
                hi = (frac > 0.0)
                qz = qn_v[z, pl.ds(r0, 16)]
                wl = qz * jnp.where(hi, 1.0 - frac, 0.0)
                wu = qz * frac
                ub_i = lb_i + hi.astype(jnp.int32)
                g_l = plsc.load_gather(nl_v, [lb_i, ridx])
                g_u = plsc.load_gather(nl_v, [ub_i, ridx])
                acc_a = acc_a + wl * g_l
                acc_b = acc_b + wu * g_u
            return acc_a, acc_b

        pa, pb = lax.fori_loop(0, SC_CHUNK // 16, grp,
                               (jnp.zeros((16,), f32), jnp.zeros((16,), f32)))
        acc_v[...] += pa + pb
    pltpu.sync_copy(acc_v, out_hbm.at[pl.ds(wid * 16, 16)])


def kernel(b_s, b_ns, b_r, b_d, W1, b1, W2, b2, TW1, Tb1, TW2, Tb2, b_a):
    B, S = b_s.shape
    H = W1.shape[1]
    n_act = W2.shape[1] // N_ATOM
    OD = n_act * GP

    def pad_head(W, b):
        Wp = jnp.pad(W.reshape(H, n_act, N_ATOM), ((0, 0), (0, 0), (0, GP - N_ATOM)))
        bp = jnp.pad(b.reshape(n_act, N_ATOM), ((0, 0), (0, GP - N_ATOM)),
                     constant_values=NEG)
        return Wp.reshape(H, OD).T, bp.reshape(OD, 1)

    W2T, b2c = pad_head(W2, b2)
    TW2T, Tb2c = pad_head(TW2, Tb2)
    a2 = b_a.reshape(1, B).astype(jnp.int32)

    grp = jnp.arange(OD) // GP
    atom = jnp.arange(OD) % GP
    vvals = (atom * V_STEP + V_MIN).astype(jnp.float32)
    Pmat = (grp[:, None] == jnp.arange(n_act)[None, :]).astype(jnp.float32)
    ind = (jnp.arange(n_act)[:, None] == grp[None, :]).astype(jnp.float32)
    Mmat = jnp.concatenate([ind, ind * vvals[None, :]], axis=0)

    BB = 1024
    BH = B // NSPLIT
    grid = BH // BB

    tc_call = pl.pallas_call(
        functools.partial(_c51_tc_block, n_act=n_act),
        grid=(grid,),
        in_specs=[
            pl.BlockSpec((BB, S), lambda i: (i, 0)),
            pl.BlockSpec((BB, S), lambda i: (i, 0)),
            pl.BlockSpec((1, BB), lambda i: (0, i)),
            pl.BlockSpec((S, H), lambda i: (0, 0)),
            pl.BlockSpec((1, H), lambda i: (0, 0)),
            pl.BlockSpec((OD, H), lambda i: (0, 0)),
            pl.BlockSpec((OD, 1), lambda i: (0, 0)),
            pl.BlockSpec((S, H), lambda i: (0, 0)),
            pl.BlockSpec((1, H), lambda i: (0, 0)),
            pl.BlockSpec((OD, H), lambda i: (0, 0)),
            pl.BlockSpec((OD, 1), lambda i: (0, 0)),
            pl.BlockSpec((OD, n_act), lambda i: (0, 0)),
            pl.BlockSpec((2 * n_act, OD), lambda i: (0, 0)),
        ],
        out_specs=[
            pl.BlockSpec((GP, BB), lambda i: (0, i)),
            pl.BlockSpec((GP, BB), lambda i: (0, i)),
        ],
        out_shape=[
            jax.ShapeDtypeStruct((GP, BH), jnp.float32),
            jax.ShapeDtypeStruct((GP, BH), jnp.float32),
        ],
    )

    sc_loss = functools.partial(
        pl.kernel,
        mesh=plsc.VectorSubcoreMesh(core_axis_name="c", subcore_axis_name="s"),
        compiler_params=pltpu.CompilerParams(needs_layout_passes=False),
        out_type=jax.ShapeDtypeStruct((NW * 16,), jnp.float32),
        scratch_types=[
            pltpu.VMEM((GP, SC_CHUNK), jnp.float32),
            pltpu.VMEM((GP, SC_CHUNK), jnp.float32),
            pltpu.VMEM((SC_CHUNK,), jnp.float32),
            pltpu.VMEM((SC_CHUNK,), jnp.float32),
            pltpu.VMEM((16,), jnp.float32),
        ],
    )(_sc_project)

    total = jnp.zeros((), jnp.float32)
    for p in range(NSPLIT):
        lo = p * BH
        qnT, nlT = tc_call(
            lax.slice_in_dim(b_s, lo, lo + BH, axis=0),
            lax.slice_in_dim(b_ns, lo, lo + BH, axis=0),
            lax.slice_in_dim(a2, lo, lo + BH, axis=1),
            W1, b1.reshape(1, H), W2T, b2c,
            TW1, Tb1.reshape(1, H), TW2T, Tb2c,
            Pmat, Mmat)
        partials = sc_loss(qnT, nlT,
                           lax.slice_in_dim(b_r, lo, lo + BH, axis=0),
                           lax.slice_in_dim(b_d, lo, lo + BH, axis=0))
        total = total + jnp.sum(partials)
    return total / (B * N_ATOM)


# speedup vs baseline: 58.3178x; 1.0414x over previous
import functools

import jax
import jax.numpy as jnp
from jax import lax
from jax.experimental import pallas as pl
from jax.experimental.pallas import tpu as pltpu
from jax.experimental.pallas import tpu_sc as plsc

V_MIN = -10.0
V_MAX = 10.0
N_ATOM = 51
GAMMA = 0.99
V_STEP = (V_MAX - V_MIN) / (N_ATOM - 1)
GP = 56
NEG = -1e30
NW = 32
SC_CHUNK = 512
NSPLIT = 4


def _tree_select(groups, bidx):
    level = list(groups)
    bit = 0
    while len(level) > 1:
        mask = ((bidx >> bit) & 1) == 1
        level = [jnp.where(mask, level[2 * k + 1], level[2 * k])
                 for k in range(len(level) // 2)]
        bit += 1
    return level[0]


def _c51_tc_block(x_ref, n_ref, a_ref,
                  W1_ref, b1_ref, W2T_ref, b2_ref,
                  TW1_ref, Tb1_ref, TW2T_ref, Tb2_ref,
                  P_ref, M_ref,
                  qn_ref, nl_ref, *, n_act):
    f32 = jnp.float32
    dn_nat = (((1,), (0,)), ((), ()))
    dn_tr = (((1,), (1,)), ((), ()))

    hn = jnp.maximum(
        lax.dot_general(n_ref[...], TW1_ref[...], dn_nat,
                        preferred_element_type=f32) + Tb1_ref[...], 0.0)
    lt = lax.dot_general(TW2T_ref[...], hn, dn_tr,
                         preferred_element_type=f32) + Tb2_ref[...]

    h = jnp.maximum(
        lax.dot_general(x_ref[...], W1_ref[...], dn_nat,
                        preferred_element_type=f32) + b1_ref[...], 0.0)
    le = lax.dot_general(W2T_ref[...], h, dn_tr,
                         preferred_element_type=f32) + b2_ref[...]

    maxes = [jnp.max(lt[a * GP:(a + 1) * GP, :], axis=0, keepdims=True)
             for a in range(n_act)]
    mstack = jnp.concatenate(maxes, axis=0)
    mfull = jnp.dot(P_ref[...], mstack, preferred_element_type=f32)
    e = jnp.exp(lt - mfull)
    S2 = jnp.dot(M_ref[...], e, preferred_element_type=f32)
    s_rows = S2[0:n_act, :]
    w_rows = S2[n_act:2 * n_act, :]
    means = w_rows / s_rows

    mmax = jnp.max(means, axis=0, keepdims=True)
    i16 = jax.lax.broadcasted_iota(jnp.int32, (n_act, 1), 0)
    bidx = jnp.min(jnp.where(means == mmax, i16, n_act), axis=0,
                   keepdims=True)

    e_sel = _tree_select([e[a * GP:(a + 1) * GP, :] for a in range(n_act)],
                         bidx)
    s_sel = _tree_select([s_rows[a:a + 1, :] for a in range(n_act)], bidx)
    qn_ref[...] = e_sel / s_sel

    aidx = a_ref[...]
    ce = _tree_select([le[a * GP:(a + 1) * GP, :] for a in range(n_act)],
                      aidx)
    m = jnp.max(ce, axis=0, keepdims=True)
    ex = jnp.exp(ce - m)
    qe = ex / jnp.sum(ex, axis=0, keepdims=True)
    nl_ref[...] = -jnp.log(qe + f32(1e-8))


def _sc_project(qn_hbm, nl_hbm, r_hbm, d_hbm, out_hbm,
                qn_v, nl_v, r_v, d_v, acc_v):
    f32 = jnp.float32
    wid = lax.axis_index("s") * 2 + lax.axis_index("c")
    rows_per_w = qn_hbm.shape[1] // NW
    base = wid * rows_per_w
    lane = lax.broadcasted_iota(jnp.int32, (16,), 0)

    acc_v[...] = jnp.zeros((16,), f32)
    for c in range(rows_per_w // SC_CHUNK):
        c0 = base + c * SC_CHUNK
        pltpu.sync_copy(qn_hbm.at[0:GP, pl.ds(c0, SC_CHUNK)], qn_v)
        pltpu.sync_copy(nl_hbm.at[0:GP, pl.ds(c0, SC_CHUNK)], nl_v)
        pltpu.sync_copy(r_hbm.at[pl.ds(c0, SC_CHUNK)], r_v)
        pltpu.sync_copy(d_hbm.at[pl.ds(c0, SC_CHUNK)], d_v)

        def grp(g, accs):
            acc_a, acc_b = accs
            r0 = g * 16
            rr = r_v[pl.ds(r0, 16)]
            sc = f32(GAMMA) * (1.0 - d_v[pl.ds(r0, 16)])
            p0 = (rr - f32(V_MIN) + sc * f32(V_MIN)) * f32(1.0 / V_STEP)
            ridx = r0 + lane
            for z in range(N_ATOM):
                pos = jnp.clip(p0 + sc * f32(z), f32(0.0), f32(N_ATOM - 1))
                lb_i = pos.astype(jnp.int32)
                frac = pos - lb_i.astype(f32)
                hi = (frac > 0.0)
                qz = qn_v[z, pl.ds(r0, 16)]
                wl = qz * jnp.where(hi, 1.0 - frac, 0.0)
                wu = qz * frac
                ub_i = lb_i + hi.astype(jnp.int32)
                g_l = plsc.load_gather(nl_v, [lb_i, ridx])
                g_u = plsc.load_gather(nl_v, [ub_i, ridx])
                acc_a = acc_a + wl * g_l
                acc_b = acc_b + wu * g_u
            return acc_a, acc_b

        pa, pb = lax.fori_loop(0, SC_CHUNK // 16, grp,
                               (jnp.zeros((16,), f32), jnp.zeros((16,), f32)))
        acc_v[...] += pa + pb
    pltpu.sync_copy(acc_v, out_hbm.at[pl.ds(wid * 16, 16)])


def kernel(b_s, b_ns, b_r, b_d, W1, b1, W2, b2, TW1, Tb1, TW2, Tb2, b_a):
    B, S = b_s.shape
    H = W1.shape[1]
    n_act = W2.shape[1] // N_ATOM
    OD = n_act * GP

    def pad_head(W, b):
        Wp = jnp.pad(W.reshape(H, n_act, N_ATOM), ((0, 0), (0, 0), (0, GP - N_ATOM)))
        bp = jnp.pad(b.reshape(n_act, N_ATOM), ((0, 0), (0, GP - N_ATOM)),
                     constant_values=NEG)
        return Wp.reshape(H, OD).T, bp.reshape(OD, 1)

    W2T, b2c = pad_head(W2, b2)
    TW2T, Tb2c = pad_head(TW2, Tb2)
    a2 = b_a.reshape(1, B).astype(jnp.int32)

    grp = jnp.arange(OD) // GP
    atom = jnp.arange(OD) % GP
    vvals = (atom * V_STEP + V_MIN).astype(jnp.float32)
    Pmat = (grp[:, None] == jnp.arange(n_act)[None, :]).astype(jnp.float32)
    ind = (jnp.arange(n_act)[:, None] == grp[None, :]).astype(jnp.float32)
    Mmat = jnp.concatenate([ind, ind * vvals[None, :]], axis=0)

    BB = 1024
    BH = B // NSPLIT
    grid = BH // BB

    tc_call = pl.pallas_call(
        functools.partial(_c51_tc_block, n_act=n_act),
        grid=(grid,),
        in_specs=[
            pl.BlockSpec((BB, S), lambda i: (i, 0)),
            pl.BlockSpec((BB, S), lambda i: (i, 0)),
            pl.BlockSpec((1, BB), lambda i: (0, i)),
            pl.BlockSpec((S, H), lambda i: (0, 0)),
            pl.BlockSpec((1, H), lambda i: (0, 0)),
            pl.BlockSpec((OD, H), lambda i: (0, 0)),
            pl.BlockSpec((OD, 1), lambda i: (0, 0)),
            pl.BlockSpec((S, H), lambda i: (0, 0)),
            pl.BlockSpec((1, H), lambda i: (0, 0)),
            pl.BlockSpec((OD, H), lambda i: (0, 0)),
            pl.BlockSpec((OD, 1), lambda i: (0, 0)),
            pl.BlockSpec((OD, n_act), lambda i: (0, 0)),
            pl.BlockSpec((2 * n_act, OD), lambda i: (0, 0)),
        ],
        out_specs=[
            pl.BlockSpec((GP, BB), lambda i: (0, i)),
            pl.BlockSpec((GP, BB), lambda i: (0, i)),
        ],
        out_shape=[
            jax.ShapeDtypeStruct((GP, BH), jnp.float32),
            jax.ShapeDtypeStruct((GP, BH), jnp.float32),
        ],
    )

    sc_loss = functools.partial(
        pl.kernel,
        mesh=plsc.VectorSubcoreMesh(core_axis_name="c", subcore_axis_name="s"),
        compiler_params=pltpu.CompilerParams(needs_layout_passes=False),
        out_type=jax.ShapeDtypeStruct((NW * 16,), jnp.float32),
        scratch_types=[
            pltpu.VMEM((GP, SC_CHUNK), jnp.float32),
            pltpu.VMEM((GP, SC_CHUNK), jnp.float32),
            pltpu.VMEM((SC_CHUNK,), jnp.float32),
            pltpu.VMEM((SC_CHUNK,), jnp.float32),
            pltpu.VMEM((16,), jnp.float32),
        ],
    )(_sc_project)

    total = jnp.zeros((), jnp.float32)
    for p in range(NSPLIT):
        lo = p * BH
        qnT, nlT = tc_call(
            lax.slice_in_dim(b_s, lo, lo + BH, axis=0),
            lax.slice_in_dim(b_ns, lo, lo + BH, axis=0),
            lax.slice_in_dim(a2, lo, lo + BH, axis=1),
            W1, b1.reshape(1, H), W2T, b2c,
            TW1, Tb1.reshape(1, H), TW2T, Tb2c,
            Pmat, Mmat)
        partials = sc_loss(qnT, nlT,
                           lax.slice_in_dim(b_r, lo, lo + BH, axis=0),
                           lax.slice_in_dim(b_d, lo, lo + BH, axis=0))
        total = total + jnp.sum(partials)
    return total / (B * N_ATOM)


# speedup vs baseline: 69.1854x; 1.1864x over previous
import functools

import jax
import jax.numpy as jnp
from jax import lax
from jax.experimental import pallas as pl
from jax.experimental.pallas import tpu as pltpu
from jax.experimental.pallas import tpu_sc as plsc

V_MIN = -10.0
V_MAX = 10.0
N_ATOM = 51
GAMMA = 0.99
V_STEP = (V_MAX - V_MIN) / (N_ATOM - 1)
GP = 56
NEG = -1e30
NW = 32
SC_CHUNK = 512
NSPLIT = 4


def _tree_select(groups, bidx):
    level = list(groups)
    bit = 0
    while len(level) > 1:
        mask = ((bidx >> bit) & 1) == 1
        level = [jnp.where(mask, level[2 * k + 1], level[2 * k])
                 for k in range(len(level) // 2)]
        bit += 1
    return level[0]


def _c51_tc_block(x_ref, n_ref, a_ref,
                  W1_ref, b1_ref, W2T_ref, b2_ref,
                  TW1_ref, Tb1_ref, TW2T_ref, Tb2_ref,
                  P_ref, M_ref,
                  qn_ref, nl_ref, *, n_act):
    f32 = jnp.float32
    dn_nat = (((1,), (0,)), ((), ()))
    dn_tr = (((1,), (1,)), ((), ()))

    hn = jnp.maximum(
        lax.dot_general(n_ref[...], TW1_ref[...], dn_nat,
                        preferred_element_type=f32) + Tb1_ref[...], 0.0)
    lt = lax.dot_general(TW2T_ref[...], hn, dn_tr,
                         preferred_element_type=f32) + Tb2_ref[...]

    h = jnp.maximum(
        lax.dot_general(x_ref[...], W1_ref[...], dn_nat,
                        preferred_element_type=f32) + b1_ref[...], 0.0)
    le = lax.dot_general(W2T_ref[...], h, dn_tr,
                         preferred_element_type=f32) + b2_ref[...]

    maxes = [jnp.max(lt[a * GP:(a + 1) * GP, :], axis=0, keepdims=True)
             for a in range(n_act)]
    mstack = jnp.concatenate(maxes, axis=0)
    mfull = jnp.dot(P_ref[...], mstack, preferred_element_type=f32)
    e = jnp.exp(lt - mfull)
    S2 = jnp.dot(M_ref[...], e, preferred_element_type=f32)
    s_rows = S2[0:n_act, :]
    w_rows = S2[n_act:2 * n_act, :]
    means = w_rows / s_rows

    mmax = jnp.max(means, axis=0, keepdims=True)
    i16 = jax.lax.broadcasted_iota(jnp.int32, (n_act, 1), 0)
    bidx = jnp.min(jnp.where(means == mmax, i16, n_act), axis=0,
                   keepdims=True)

    e_sel = _tree_select([e[a * GP:(a + 1) * GP, :] for a in range(n_act)],
                         bidx)
    s_sel = _tree_select([s_rows[a:a + 1, :] for a in range(n_act)], bidx)
    qn_ref[...] = e_sel / s_sel

    aidx = a_ref[...]
    ce = _tree_select([le[a * GP:(a + 1) * GP, :] for a in range(n_act)],
                      aidx)
    m = jnp.max(ce, axis=0, keepdims=True)
    ex = jnp.exp(ce - m)
    qe = ex / jnp.sum(ex, axis=0, keepdims=True)
    nl_ref[...] = -jnp.log(qe + f32(1e-8))


def _sc_project(qn_hbm, nl_hbm, r_hbm, d_hbm, out_hbm,
                qn_v, nl_v, r_v, d_v, acc_v, *, rdoff):
    f32 = jnp.float32
    wid = lax.axis_index("s") * 2 + lax.axis_index("c")
    rows_per_w = qn_hbm.shape[1] // NW
    base = wid * rows_per_w
    lane = lax.broadcasted_iota(jnp.int32, (16,), 0)

    acc_v[...] = jnp.zeros((16,), f32)
    for c in range(rows_per_w // SC_CHUNK):
        c0 = base + c * SC_CHUNK
        pltpu.sync_copy(qn_hbm.at[0:GP, pl.ds(c0, SC_CHUNK)], qn_v)
        pltpu.sync_copy(nl_hbm.at[0:GP, pl.ds(c0, SC_CHUNK)], nl_v)
        pltpu.sync_copy(r_hbm.at[pl.ds(rdoff + c0, SC_CHUNK)], r_v)
        pltpu.sync_copy(d_hbm.at[pl.ds(rdoff + c0, SC_CHUNK)], d_v)

        def grp(g, accs):
            acc_a, acc_b = accs
            r0 = g * 16
            rr = r_v[pl.ds(r0, 16)]
            sc = f32(GAMMA) * (1.0 - d_v[pl.ds(r0, 16)])
            p0 = (rr - f32(V_MIN) + sc * f32(V_MIN)) * f32(1.0 / V_STEP)
            ridx = r0 + lane
            for z in range(N_ATOM):
                pos = jnp.clip(p0 + sc * f32(z), f32(0.0), f32(N_ATOM - 1))
                lb_i = pos.astype(jnp.int32)
                frac = pos - lb_i.astype(f32)
                hi = (frac > 0.0)
                qz = qn_v[z, pl.ds(r0, 16)]
                wl = qz * jnp.where(hi, 1.0 - frac, 0.0)
                wu = qz * frac
                ub_i = lb_i + hi.astype(jnp.int32)
                g_l = plsc.load_gather(nl_v, [lb_i, ridx])
                g_u = plsc.load_gather(nl_v, [ub_i, ridx])
                acc_a = acc_a + wl * g_l
                acc_b = acc_b + wu * g_u
            return acc_a, acc_b

        pa, pb = lax.fori_loop(0, SC_CHUNK // 16, grp,
                               (jnp.zeros((16,), f32), jnp.zeros((16,), f32)))
        acc_v[...] += pa + pb
    pltpu.sync_copy(acc_v, out_hbm.at[pl.ds(wid * 16, 16)])


def kernel(b_s, b_ns, b_r, b_d, W1, b1, W2, b2, TW1, Tb1, TW2, Tb2, b_a):
    B, S = b_s.shape
    H = W1.shape[1]
    n_act = W2.shape[1] // N_ATOM
    OD = n_act * GP

    def pad_head(W, b):
        Wp = jnp.pad(W.reshape(H, n_act, N_ATOM), ((0, 0), (0, 0), (0, GP - N_ATOM)))
        bp = jnp.pad(b.reshape(n_act, N_ATOM), ((0, 0), (0, GP - N_ATOM)),
                     constant_values=NEG)
        return Wp.reshape(H, OD).T, bp.reshape(OD, 1)

    W2T, b2c = pad_head(W2, b2)
    TW2T, Tb2c = pad_head(TW2, Tb2)
    a2 = b_a.reshape(1, B).astype(jnp.int32)

    grp = jnp.arange(OD) // GP
    atom = jnp.arange(OD) % GP
    vvals = (atom * V_STEP + V_MIN).astype(jnp.float32)
    Pmat = (grp[:, None] == jnp.arange(n_act)[None, :]).astype(jnp.float32)
    ind = (jnp.arange(n_act)[:, None] == grp[None, :]).astype(jnp.float32)
    Mmat = jnp.concatenate([ind, ind * vvals[None, :]], axis=0)

    BB = 1024
    BH = B // NSPLIT
    grid = BH // BB

    def tc_call(p):
        off = p * grid
        return pl.pallas_call(
            functools.partial(_c51_tc_block, n_act=n_act),
            grid=(grid,),
            in_specs=[
                pl.BlockSpec((BB, S), lambda i: (off + i, 0)),
                pl.BlockSpec((BB, S), lambda i: (off + i, 0)),
                pl.BlockSpec((1, BB), lambda i: (0, off + i)),
                pl.BlockSpec((S, H), lambda i: (0, 0)),
                pl.BlockSpec((1, H), lambda i: (0, 0)),
                pl.BlockSpec((OD, H), lambda i: (0, 0)),
                pl.BlockSpec((OD, 1), lambda i: (0, 0)),
                pl.BlockSpec((S, H), lambda i: (0, 0)),
                pl.BlockSpec((1, H), lambda i: (0, 0)),
                pl.BlockSpec((OD, H), lambda i: (0, 0)),
                pl.BlockSpec((OD, 1), lambda i: (0, 0)),
                pl.BlockSpec((OD, n_act), lambda i: (0, 0)),
                pl.BlockSpec((2 * n_act, OD), lambda i: (0, 0)),
            ],
            out_specs=[
                pl.BlockSpec((GP, BB), lambda i: (0, i)),
                pl.BlockSpec((GP, BB), lambda i: (0, i)),
            ],
            out_shape=[
                jax.ShapeDtypeStruct((GP, BH), jnp.float32),
                jax.ShapeDtypeStruct((GP, BH), jnp.float32),
            ],
        )

    def sc_loss(p):
        return functools.partial(
            pl.kernel,
            mesh=plsc.VectorSubcoreMesh(core_axis_name="c",
                                        subcore_axis_name="s"),
            compiler_params=pltpu.CompilerParams(needs_layout_passes=False),
            out_type=jax.ShapeDtypeStruct((NW * 16,), jnp.float32),
            scratch_types=[
                pltpu.VMEM((GP, SC_CHUNK), jnp.float32),
                pltpu.VMEM((GP, SC_CHUNK), jnp.float32),
                pltpu.VMEM((SC_CHUNK,), jnp.float32),
                pltpu.VMEM((SC_CHUNK,), jnp.float32),
                pltpu.VMEM((16,), jnp.float32),
            ],
        )(functools.partial(_sc_project, rdoff=p * BH))

    total = jnp.zeros((), jnp.float32)
    for p in range(NSPLIT):
        qnT, nlT = tc_call(p)(
            b_s, b_ns, a2,
            W1, b1.reshape(1, H), W2T, b2c,
            TW1, Tb1.reshape(1, H), TW2T, Tb2c,
            Pmat, Mmat)
        partials = sc_loss(p)(qnT, nlT, b_r, b_d)
        total = total + jnp.sum(partials)
    return total / (B * N_ATOM)


# speedup vs baseline: 69.2279x; 1.0006x over previous
import functools

import jax
import jax.numpy as jnp
from jax import lax
from jax.experimental import pallas as pl
from jax.experimental.pallas import tpu as pltpu
from jax.experimental.pallas import tpu_sc as plsc

V_MIN = -10.0
V_MAX = 10.0
N_ATOM = 51
GAMMA = 0.99
V_STEP = (V_MAX - V_MIN) / (N_ATOM - 1)
GP = 56
NEG = -1e30
NW = 32
SC_CHUNK = 512
NSPLIT = 4


def _tree_select(groups, bidx):
    level = list(groups)
    bit = 0
    while len(level) > 1:
        mask = ((bidx >> bit) & 1) == 1
        level = [jnp.where(mask, level[2 * k + 1], level[2 * k])
                 for k in range(len(level) // 2)]
        bit += 1
    return level[0]


def _c51_tc_block(x_ref, n_ref, a_ref,
                  W1_ref, b1_ref, W2T_ref, b2_ref,
                  TW1_ref, Tb1_ref, TW2T_ref, Tb2_ref,
                  P_ref, M_ref,
                  qn_ref, nl_ref, *, n_act):
    f32 = jnp.float32
    dn_nat = (((1,), (0,)), ((), ()))
    dn_tr = (((1,), (1,)), ((), ()))

    hn = jnp.maximum(
        lax.dot_general(n_ref[...], TW1_ref[...], dn_nat,
                        preferred_element_type=f32) + Tb1_ref[...], 0.0)
    lt = lax.dot_general(TW2T_ref[...], hn, dn_tr,
                         preferred_element_type=f32) + Tb2_ref[...]

    h = jnp.maximum(
        lax.dot_general(x_ref[...], W1_ref[...], dn_nat,
                        preferred_element_type=f32) + b1_ref[...], 0.0)
    le = lax.dot_general(W2T_ref[...], h, dn_tr,
                         preferred_element_type=f32) + b2_ref[...]

    maxes = [jnp.max(lt[a * GP:(a + 1) * GP, :], axis=0, keepdims=True)
             for a in range(n_act)]
    mstack = jnp.concatenate(maxes, axis=0)
    mfull = jnp.dot(P_ref[...], mstack, preferred_element_type=f32)
    e = jnp.exp(lt - mfull)
    S2 = jnp.dot(M_ref[...], e, preferred_element_type=f32)
    s_rows = S2[0:n_act, :]
    w_rows = S2[n_act:2 * n_act, :]
    means = w_rows / s_rows

    mmax = jnp.max(means, axis=0, keepdims=True)
    i16 = jax.lax.broadcasted_iota(jnp.int32, (n_act, 1), 0)
    bidx = jnp.min(jnp.where(means == mmax, i16, n_act), axis=0,
                   keepdims=True)

    e_sel = _tree_select([e[a * GP:(a + 1) * GP, :] for a in range(n_act)],
                         bidx)
    s_sel = _tree_select([s_rows[a:a + 1, :] for a in range(n_act)], bidx)
    qn_ref[...] = e_sel / s_sel

    aidx = a_ref[...]
    ce = _tree_select([le[a * GP:(a + 1) * GP, :] for a in range(n_act)],
                      aidx)
    m = jnp.max(ce, axis=0, keepdims=True)
    ex = jnp.exp(ce - m)
    qe = ex / jnp.sum(ex, axis=0, keepdims=True)
    nl_ref[...] = -jnp.log(qe + f32(1e-8))


def _sc_project(qn_hbm, nl_hbm, r_hbm, d_hbm, out_hbm,
                qn_v, nl_v, r_v, d_v, acc_v, *, rdoff):
    f32 = jnp.float32
    wid = lax.axis_index("s") * 2 + lax.axis_index("c")
    rows_per_w = qn_hbm.shape[1] // NW
    base = wid * rows_per_w
    lane = lax.broadcasted_iota(jnp.int32, (16,), 0)

    acc_v[...] = jnp.zeros((16,), f32)
    for c in range(rows_per_w // SC_CHUNK):
        c0 = base + c * SC_CHUNK
        pltpu.sync_copy(qn_hbm.at[0:GP, pl.ds(c0, SC_CHUNK)], qn_v)
        pltpu.sync_copy(nl_hbm.at[0:GP, pl.ds(c0, SC_CHUNK)], nl_v)
        pltpu.sync_copy(r_hbm.at[pl.ds(rdoff + c0, SC_CHUNK)], r_v)
        pltpu.sync_copy(d_hbm.at[pl.ds(rdoff + c0, SC_CHUNK)], d_v)

        def grp(g, accs):
            acc_a, acc_b = accs
            r0 = g * 16
            rr = r_v[pl.ds(r0, 16)]
            sc = f32(GAMMA) * (1.0 - d_v[pl.ds(r0, 16)])
            p0 = (rr - f32(V_MIN) + sc * f32(V_MIN)) * f32(1.0 / V_STEP)
            ridx = r0 + lane
            for z in range(N_ATOM):
                pos = jnp.clip(p0 + sc * f32(z), f32(0.0), f32(N_ATOM - 1))
                lb_i = pos.astype(jnp.int32)
                frac = pos - lb_i.astype(f32)
                hi = (frac > 0.0)
                qz = qn_v[z, pl.ds(r0, 16)]
                wl = qz * jnp.where(hi, 1.0 - frac, 0.0)
                wu = qz * frac
                ub_i = lb_i + hi.astype(jnp.int32)
                g_l = plsc.load_gather(nl_v, [lb_i, ridx])
                g_u = plsc.load_gather(nl_v, [ub_i, ridx])
                acc_a = acc_a + wl * g_l
                acc_b = acc_b + wu * g_u
            return acc_a, acc_b

        pa, pb = lax.fori_loop(0, SC_CHUNK // 16, grp,
                               (jnp.zeros((16,), f32), jnp.zeros((16,), f32)))
        acc_v[...] += pa + pb
    pltpu.sync_copy(acc_v, out_hbm.at[pl.ds(wid * 16, 16)])


def kernel(b_s, b_ns, b_r, b_d, W1, b1, W2, b2, TW1, Tb1, TW2, Tb2, b_a):
    B, S = b_s.shape
    H = W1.shape[1]
    n_act = W2.shape[1] // N_ATOM
    OD = n_act * GP

    def pad_head(W, b):
        Wp = jnp.pad(W.reshape(H, n_act, N_ATOM), ((0, 0), (0, 0), (0, GP - N_ATOM)))
        bp = jnp.pad(b.reshape(n_act, N_ATOM), ((0, 0), (0, GP - N_ATOM)),
                     constant_values=NEG)
        return Wp.reshape(H, OD).T, bp.reshape(OD, 1)

    W2T, b2c = pad_head(W2, b2)
    TW2T, Tb2c = pad_head(TW2, Tb2)
    a2 = b_a.reshape(1, B).astype(jnp.int32)

    grp = jnp.arange(OD) // GP
    atom = jnp.arange(OD) % GP
    vvals = (atom * V_STEP + V_MIN).astype(jnp.float32)
    Pmat = (grp[:, None] == jnp.arange(n_act)[None, :]).astype(jnp.float32)
    ind = (jnp.arange(n_act)[:, None] == grp[None, :]).astype(jnp.float32)
    Mmat = jnp.concatenate([ind, ind * vvals[None, :]], axis=0)

    BB = 1024
    BH = B // NSPLIT
    grid = BH // BB

    def tc_call(p):
        off = p * grid
        return pl.pallas_call(
            functools.partial(_c51_tc_block, n_act=n_act),
            grid=(grid,),
            in_specs=[
                pl.BlockSpec((BB, S), lambda i: (off + i, 0)),
                pl.BlockSpec((BB, S), lambda i: (off + i, 0)),
                pl.BlockSpec((1, BB), lambda i: (0, off + i)),
                pl.BlockSpec((S, H), lambda i: (0, 0)),
                pl.BlockSpec((1, H), lambda i: (0, 0)),
                pl.BlockSpec((OD, H), lambda i: (0, 0)),
                pl.BlockSpec((OD, 1), lambda i: (0, 0)),
                pl.BlockSpec((S, H), lambda i: (0, 0)),
                pl.BlockSpec((1, H), lambda i: (0, 0)),
                pl.BlockSpec((OD, H), lambda i: (0, 0)),
                pl.BlockSpec((OD, 1), lambda i: (0, 0)),
                pl.BlockSpec((OD, n_act), lambda i: (0, 0)),
                pl.BlockSpec((2 * n_act, OD), lambda i: (0, 0)),
            ],
            out_specs=[
                pl.BlockSpec((GP, BB), lambda i: (0, i)),
                pl.BlockSpec((GP, BB), lambda i: (0, i)),
            ],
            out_shape=[
                jax.ShapeDtypeStruct((GP, BH), jnp.float32),
                jax.ShapeDtypeStruct((GP, BH), jnp.float32),
            ],
        )

    def sc_loss(p):
        return functools.partial(
            pl.kernel,
            mesh=plsc.VectorSubcoreMesh(core_axis_name="c",
                                        subcore_axis_name="s"),
            compiler_params=pltpu.CompilerParams(needs_layout_passes=False),
            out_type=jax.ShapeDtypeStruct((NW * 16,), jnp.float32),
            scratch_types=[
                pltpu.VMEM((GP, SC_CHUNK), jnp.float32),
                pltpu.VMEM((GP, SC_CHUNK), jnp.float32),
                pltpu.VMEM((SC_CHUNK,), jnp.float32),
                pltpu.VMEM((SC_CHUNK,), jnp.float32),
                pltpu.VMEM((16,), jnp.float32),
            ],
        )(functools.partial(_sc_project, rdoff=p * BH))

    outs = []
    for p in range(NSPLIT):
        outs.append(tc_call(p)(
            b_s, b_ns, a2,
            W1, b1.reshape(1, H), W2T, b2c,
            TW1, Tb1.reshape(1, H), TW2T, Tb2c,
            Pmat, Mmat))
    total = jnp.zeros((), jnp.float32)
    for p in range(NSPLIT):
        qnT, nlT = outs[p]
        total = total + jnp.sum(sc_loss(p)(qnT, nlT, b_r, b_d))
    return total / (B * N_ATOM)
